# Initial kernel scaffold; baseline (speedup 1.0000x reference)
import jax
import jax.numpy as jnp
from jax.experimental import pallas as pl


def kernel(x, iInd, jInd):
    raise NotImplementedError("write your pallas kernel here")


# speedup vs baseline: 156.0667x; 156.0667x over previous
import functools

import jax
import jax.numpy as jnp
from jax import lax
from jax.experimental import pallas as pl
from jax.experimental.pallas import tpu as pltpu
from jax.experimental.pallas import tpu_sc as plsc

N_NODES = 100000
N_EDGES = 1600000
C_REAL = 12
C_PAD = 16
NC = 2
NS = 16
L = 16
CHUNK = 128
E_PAD = ((N_EDGES + NS * CHUNK - 1) // (NS * CHUNK)) * (NS * CHUNK)
EPT = E_PAD // NS
NCHUNK = EPT // CHUNK
N_PADN = 100352
ROWS_PER_TILE = N_PADN // NS
RD = 784


def _body(xh, ii, jj, out,
          ii_raw, jj_raw, ii_off, jj_off, xi, xj, gp, gn, rw, acc,
          sem_i, sem_j):
    c = lax.axis_index("c")
    s = lax.axis_index("s")

    zero = jnp.zeros((L,), jnp.float32)

    def zrow(r, carry):
        rw[r, :] = zero
        return carry

    lax.fori_loop(0, RD, zrow, 0)
    row_base = s * ROWS_PER_TILE
    for t in range(ROWS_PER_TILE // RD):
        pltpu.sync_copy(rw, acc.at[pl.ds(row_base + t * RD, RD)])
    plsc.subcore_barrier()

    offv = jnp.full((L,), c * N_PADN, jnp.int32)
    ebase = s * EPT

    def chunk(g, carry):
        b = ebase + g * CHUNK
        pltpu.sync_copy(ii.at[pl.ds(b, CHUNK)], ii_raw)
        pltpu.sync_copy(jj.at[pl.ds(b, CHUNK)], jj_raw)
        for k in range(CHUNK // L):
            sl = pl.ds(k * L, L)
            ii_off[sl] = ii_raw[sl] + offv
            jj_off[sl] = jj_raw[sl] + offv
        cp_i = pltpu.async_copy(xh.at[ii_off], xi, sem_i)
        cp_j = pltpu.async_copy(xh.at[jj_off], xj, sem_j)
        cp_i.wait()
        cp_j.wait()

        def row(r, carry2):
            a = xi[r, :]
            b2 = xj[r, :]
            gp[r, :] = a - b2
            gn[r, :] = b2 - a
            return carry2

        lax.fori_loop(0, CHUNK, row, 0)
        pltpu.sync_copy(gp, acc.at[ii_raw], add=True)
        pltpu.sync_copy(gn, acc.at[jj_raw], add=True)
        return carry

    lax.fori_loop(0, NCHUNK, chunk, 0)
    plsc.subcore_barrier()

    out_base = c * N_PADN + s * ROWS_PER_TILE
    for t in range(ROWS_PER_TILE // RD):
        pltpu.sync_copy(acc.at[pl.ds(row_base + t * RD, RD)], rw)
        pltpu.sync_copy(rw, out.at[pl.ds(out_base + t * RD, RD)])


_graph_lap = functools.partial(
    pl.kernel,
    out_type=jax.ShapeDtypeStruct((NC * N_PADN, C_PAD), jnp.float32),
    mesh=plsc.VectorSubcoreMesh(core_axis_name="c", subcore_axis_name="s"),
    compiler_params=pltpu.CompilerParams(use_tc_tiling_on_sc=False),
    scratch_types=[
        pltpu.VMEM((CHUNK,), jnp.int32),
        pltpu.VMEM((CHUNK,), jnp.int32),
        pltpu.VMEM((CHUNK,), jnp.int32),
        pltpu.VMEM((CHUNK,), jnp.int32),
        pltpu.VMEM((CHUNK, C_PAD), jnp.float32),
        pltpu.VMEM((CHUNK, C_PAD), jnp.float32),
        pltpu.VMEM((CHUNK, C_PAD), jnp.float32),
        pltpu.VMEM((CHUNK, C_PAD), jnp.float32),
        pltpu.VMEM((RD, C_PAD), jnp.float32),
        pltpu.VMEM_SHARED((N_PADN, C_PAD), jnp.float32),
        pltpu.SemaphoreType.DMA,
        pltpu.SemaphoreType.DMA,
    ],
)(_body)


def kernel(x, iInd, jInd):
    ii = iInd.astype(jnp.int32)
    jj = jInd.astype(jnp.int32)
    xt = jnp.transpose(x.reshape(24, N_NODES))
    zpad = jnp.zeros((N_NODES, C_PAD - C_REAL), jnp.float32)
    zrows = jnp.zeros((N_PADN - N_NODES, C_PAD), jnp.float32)
    xh = jnp.concatenate(
        [
            jnp.concatenate([xt[:, :C_REAL], zpad], axis=1),
            zrows,
            jnp.concatenate([xt[:, C_REAL:], zpad], axis=1),
            zrows,
        ],
        axis=0,
    )
    epad = jnp.zeros((E_PAD - N_EDGES,), jnp.int32)
    ii_p = jnp.concatenate([ii, epad])
    jj_p = jnp.concatenate([jj, epad])
    o = _graph_lap(xh, ii_p, jj_p)
    full = jnp.concatenate(
        [o[:N_NODES, :C_REAL], o[N_PADN:N_PADN + N_NODES, :C_REAL]], axis=1)
    return jnp.transpose(full).reshape(1, 8, 3, N_NODES)
